# Initial kernel scaffold; baseline (speedup 1.0000x reference)
import jax
import jax.numpy as jnp
from jax.experimental import pallas as pl


def kernel(x, edge_index, W1, b1, W2, b2):
    raise NotImplementedError("write your pallas kernel here")


# speedup vs baseline: 15.7349x; 15.7349x over previous
import functools

import jax
import jax.numpy as jnp
from jax import lax
from jax.experimental import pallas as pl
from jax.experimental.pallas import tpu as pltpu
from jax.experimental.pallas import tpu_sc as plsc

N = 10000
E = 320000
D = 128
H = 128
CP = 128

NC = 2
NS = 16
NW = NC * NS
CHUNK = 128
CPW = -(-E // (NW * CHUNK))
E_PAD = NW * CPW * CHUNK
PAD_ROWS = 240
NP = N + PAD_ROWS
RPT = NP // NS

_MESH = plsc.VectorSubcoreMesh(core_axis_name="c", subcore_axis_name="s")


def _sc_deg(dst_p, zeros1):
    @functools.partial(
        pl.kernel,
        mesh=_MESH,
        out_type=jax.ShapeDtypeStruct((NC, NP), jnp.float32),
        scratch_types=[
            pltpu.VMEM((CHUNK,), jnp.int32),
            pltpu.VMEM((CHUNK,), jnp.float32),
            pltpu.VMEM((RPT,), jnp.float32),
            pltpu.VMEM_SHARED((NP,), jnp.float32),
            pltpu.SemaphoreType.DMA,
        ],
    )
    def deg_kernel(dst_hbm, zeros_hbm, out_hbm, idx_v, ones_v, row_v, acc, sem):
        cid = lax.axis_index("c")
        sid = lax.axis_index("s")
        wid = cid * NS + sid
        for i in range(CHUNK // 16):
            ones_v[pl.ds(i * 16, 16)] = jnp.ones((16,), jnp.float32)
        pltpu.sync_copy(zeros_hbm.at[pl.ds(sid * RPT, RPT)], row_v)
        pltpu.sync_copy(row_v, acc.at[pl.ds(sid * RPT, RPT)])
        plsc.subcore_barrier()

        def body(j, carry):
            base = (wid * CPW + j) * CHUNK
            pltpu.sync_copy(dst_hbm.at[pl.ds(base, CHUNK)], idx_v)
            pltpu.sync_copy(ones_v, acc.at[idx_v], add=True)
            return carry

        lax.fori_loop(0, CPW, body, 0)
        plsc.subcore_barrier()
        pltpu.sync_copy(acc.at[pl.ds(sid * RPT, RPT)], row_v)
        pltpu.sync_copy(row_v, out_hbm.at[cid, pl.ds(sid * RPT, RPT)])

    return deg_kernel(dst_p, zeros1)


def _make_sc_agg(F):
    @functools.partial(
        pl.kernel,
        mesh=_MESH,
        out_type=jax.ShapeDtypeStruct((NC, NP, F), jnp.float32),
        scratch_types=[
            pltpu.VMEM((CHUNK,), jnp.int32),
            pltpu.VMEM((CHUNK,), jnp.int32),
            pltpu.VMEM((CHUNK, F), jnp.float32),
            pltpu.VMEM_SHARED((NP, F), jnp.float32),
            pltpu.SemaphoreType.DMA,
        ],
    )
    def agg_kernel(table_hbm, src_hbm, dst_hbm, zeros_hbm, out_hbm,
                   sidx, didx, rows, acc, sem):
        cid = lax.axis_index("c")
        sid = lax.axis_index("s")
        wid = cid * NS + sid
        for i in range(RPT // CHUNK):
            r0 = sid * RPT + i * CHUNK
            pltpu.sync_copy(zeros_hbm.at[pl.ds(r0, CHUNK)], rows)
            pltpu.sync_copy(rows, acc.at[pl.ds(r0, CHUNK)])
        plsc.subcore_barrier()

        def body(j, carry):
            base = (wid * CPW + j) * CHUNK
            pltpu.sync_copy(src_hbm.at[pl.ds(base, CHUNK)], sidx)
            pltpu.sync_copy(dst_hbm.at[pl.ds(base, CHUNK)], didx)
            pltpu.async_copy(table_hbm.at[sidx], rows, sem).wait()
            pltpu.sync_copy(rows, acc.at[didx], add=True)
            return carry

        lax.fori_loop(0, CPW, body, 0)
        plsc.subcore_barrier()
        for i in range(RPT // CHUNK):
            r0 = sid * RPT + i * CHUNK
            pltpu.sync_copy(acc.at[pl.ds(r0, CHUNK)], rows)
            pltpu.sync_copy(rows, out_hbm.at[cid, pl.ds(r0, CHUNK)])

    return agg_kernel


def _tc_prep(x, W1, deg_t):
    def body(x_ref, w_ref, d_ref, g_ref, dis_ref):
        h = jnp.dot(x_ref[...], w_ref[...], preferred_element_type=jnp.float32)
        d = d_ref[...]
        dis = lax.rsqrt(d[:, 0:1] + d[:, 1:2] + 1.0)
        dis_ref[...] = dis
        g_ref[pl.ds(0, N), :] = h * dis[:N]
        g_ref[pl.ds(N, PAD_ROWS), :] = jnp.zeros((PAD_ROWS, D), jnp.float32)

    return pl.pallas_call(
        body,
        out_shape=(
            jax.ShapeDtypeStruct((NP, D), jnp.float32),
            jax.ShapeDtypeStruct((NP, 1), jnp.float32),
        ),
    )(x, W1, deg_t)


def _tc_mid(acc1, g, dis, b1r, W2p):
    def body(a_ref, g_ref, dis_ref, b_ref, w_ref, z_ref):
        s = a_ref[0] + a_ref[1] + g_ref[...]
        y = s * dis_ref[...] + b_ref[...]
        h1 = jnp.maximum(y, 0.0)
        z_ref[...] = jnp.dot(h1, w_ref[...],
                             preferred_element_type=jnp.float32) * dis_ref[...]

    return pl.pallas_call(
        body,
        out_shape=jax.ShapeDtypeStruct((NP, CP), jnp.float32),
    )(acc1, g, dis, b1r, W2p)


def _tc_out(acc2, z2, dis, b2r):
    def body(a_ref, z_ref, dis_ref, b_ref, o_ref):
        s = (a_ref[0] + a_ref[1] + z_ref[...]) * dis_ref[...]
        y = s[:N, 0:2] + b_ref[...]
        m = jnp.max(y, axis=1, keepdims=True)
        e = jnp.exp(y - m)
        lse = m + jnp.log(e[:, 0:1] + e[:, 1:2])
        o_ref[...] = y - lse

    return pl.pallas_call(
        body,
        out_shape=jax.ShapeDtypeStruct((N, 2), jnp.float32),
    )(acc2, z2, dis, b2r)


_agg_wide = _make_sc_agg(D)
_agg_narrow = _make_sc_agg(CP)


def kernel(x, edge_index, W1, b1, W2, b2):
    pad = N + (jnp.arange(E_PAD - E, dtype=jnp.int32) % PAD_ROWS)
    src_p = jnp.concatenate([edge_index[0], pad])
    dst_p = jnp.concatenate([edge_index[1], pad])
    zeros1 = jnp.zeros((NP,), jnp.float32)
    zerosD = jnp.zeros((NP, D), jnp.float32)
    zerosC = jnp.zeros((NP, CP), jnp.float32)
    W2p = jnp.pad(W2, ((0, 0), (0, CP - W2.shape[1])))
    b1r = b1.reshape(1, H)
    b2r = b2.reshape(1, 2)

    deg2 = _sc_deg(dst_p, zeros1)
    g, dis = _tc_prep(x, W1, deg2.T)
    acc1 = _agg_wide(g, src_p, dst_p, zerosD)
    z2 = _tc_mid(acc1, g, dis, b1r, W2p)
    acc2 = _agg_narrow(z2, src_p, dst_p, zerosC)
    return _tc_out(acc2, z2, dis, b2r)


# speedup vs baseline: 20.7807x; 1.3207x over previous
import functools

import jax
import jax.numpy as jnp
from jax import lax
from jax.experimental import pallas as pl
from jax.experimental.pallas import tpu as pltpu
from jax.experimental.pallas import tpu_sc as plsc

N = 10000
E = 320000
D = 128
H = 128
CP = 128

NC = 2
NS = 16
NW = NC * NS
CHUNK = 128
CPW = 2 * -(-E // (NW * CHUNK * 2))
E_PAD = NW * CPW * CHUNK
PAD_ROWS = 240
NP = N + PAD_ROWS
RPT = NP // NS

_MESH = plsc.VectorSubcoreMesh(core_axis_name="c", subcore_axis_name="s")


def _sc_deg(dst_p, zeros1):
    @functools.partial(
        pl.kernel,
        mesh=_MESH,
        out_type=jax.ShapeDtypeStruct((NC, NP), jnp.float32),
        scratch_types=[
            pltpu.VMEM((CHUNK,), jnp.int32),
            pltpu.VMEM((CHUNK,), jnp.float32),
            pltpu.VMEM((RPT,), jnp.float32),
            pltpu.VMEM_SHARED((NP,), jnp.float32),
            pltpu.SemaphoreType.DMA,
        ],
    )
    def deg_kernel(dst_hbm, zeros_hbm, out_hbm, idx_v, ones_v, row_v, acc, sem):
        cid = lax.axis_index("c")
        sid = lax.axis_index("s")
        wid = cid * NS + sid
        for i in range(CHUNK // 16):
            ones_v[pl.ds(i * 16, 16)] = jnp.ones((16,), jnp.float32)
        pltpu.sync_copy(zeros_hbm.at[pl.ds(sid * RPT, RPT)], row_v)
        pltpu.sync_copy(row_v, acc.at[pl.ds(sid * RPT, RPT)])
        plsc.subcore_barrier()

        def body(j, carry):
            base = (wid * CPW + j) * CHUNK
            pltpu.sync_copy(dst_hbm.at[pl.ds(base, CHUNK)], idx_v)
            pltpu.sync_copy(ones_v, acc.at[idx_v], add=True)
            return carry

        lax.fori_loop(0, CPW, body, 0)
        plsc.subcore_barrier()
        pltpu.sync_copy(acc.at[pl.ds(sid * RPT, RPT)], row_v)
        pltpu.sync_copy(row_v, out_hbm.at[cid, pl.ds(sid * RPT, RPT)])

    return deg_kernel(dst_p, zeros1)


def _make_sc_agg(F):
    @functools.partial(
        pl.kernel,
        mesh=_MESH,
        out_type=jax.ShapeDtypeStruct((NC, NP, F), jnp.float32),
        scratch_types=[
            pltpu.VMEM((CHUNK,), jnp.int32),
            pltpu.VMEM((CHUNK,), jnp.int32),
            pltpu.VMEM((CHUNK,), jnp.int32),
            pltpu.VMEM((CHUNK,), jnp.int32),
            pltpu.VMEM((CHUNK, F), jnp.float32),
            pltpu.VMEM((CHUNK, F), jnp.float32),
            pltpu.VMEM_SHARED((NP, F), jnp.float32),
            pltpu.SemaphoreType.DMA,
            pltpu.SemaphoreType.DMA,
        ],
    )
    def agg_kernel(table_hbm, src_hbm, dst_hbm, zeros_hbm, out_hbm,
                   sidx0, sidx1, didx0, didx1, rows0, rows1, acc, sem0, sem1):
        cid = lax.axis_index("c")
        sid = lax.axis_index("s")
        wid = cid * NS + sid
        for i in range(RPT // CHUNK):
            r0 = sid * RPT + i * CHUNK
            pltpu.sync_copy(zeros_hbm.at[pl.ds(r0, CHUNK)], rows0)
            pltpu.sync_copy(rows0, acc.at[pl.ds(r0, CHUNK)])
        plsc.subcore_barrier()

        def body(jj, carry):
            base0 = (wid * CPW + 2 * jj) * CHUNK
            base1 = base0 + CHUNK
            pltpu.sync_copy(src_hbm.at[pl.ds(base0, CHUNK)], sidx0)
            h0 = pltpu.async_copy(table_hbm.at[sidx0], rows0, sem0)
            pltpu.sync_copy(dst_hbm.at[pl.ds(base0, CHUNK)], didx0)
            pltpu.sync_copy(src_hbm.at[pl.ds(base1, CHUNK)], sidx1)
            h0.wait()
            h1 = pltpu.async_copy(table_hbm.at[sidx1], rows1, sem1)
            pltpu.sync_copy(rows0, acc.at[didx0], add=True)
            pltpu.sync_copy(dst_hbm.at[pl.ds(base1, CHUNK)], didx1)
            h1.wait()
            pltpu.sync_copy(rows1, acc.at[didx1], add=True)
            return carry

        lax.fori_loop(0, CPW // 2, body, 0)
        plsc.subcore_barrier()
        for i in range(RPT // CHUNK):
            r0 = sid * RPT + i * CHUNK
            pltpu.sync_copy(acc.at[pl.ds(r0, CHUNK)], rows0)
            pltpu.sync_copy(rows0, out_hbm.at[cid, pl.ds(r0, CHUNK)])

    return agg_kernel


def _tc_prep(x, W1, deg_t):
    def body(x_ref, w_ref, d_ref, g_ref, dis_ref):
        h = jnp.dot(x_ref[...], w_ref[...], preferred_element_type=jnp.float32)
        d = d_ref[...]
        dis = lax.rsqrt(d[:, 0:1] + d[:, 1:2] + 1.0)
        dis_ref[...] = dis
        g_ref[pl.ds(0, N), :] = h * dis[:N]
        g_ref[pl.ds(N, PAD_ROWS), :] = jnp.zeros((PAD_ROWS, D), jnp.float32)

    return pl.pallas_call(
        body,
        out_shape=(
            jax.ShapeDtypeStruct((NP, D), jnp.float32),
            jax.ShapeDtypeStruct((NP, 1), jnp.float32),
        ),
    )(x, W1, deg_t)


def _tc_mid(acc1, g, dis, b1r, W2p):
    def body(a_ref, g_ref, dis_ref, b_ref, w_ref, z_ref):
        s = a_ref[0] + a_ref[1] + g_ref[...]
        y = s * dis_ref[...] + b_ref[...]
        h1 = jnp.maximum(y, 0.0)
        z_ref[...] = jnp.dot(h1, w_ref[...],
                             preferred_element_type=jnp.float32) * dis_ref[...]

    return pl.pallas_call(
        body,
        out_shape=jax.ShapeDtypeStruct((NP, CP), jnp.float32),
    )(acc1, g, dis, b1r, W2p)


def _tc_out(acc2, z2, dis, b2r):
    def body(a_ref, z_ref, dis_ref, b_ref, o_ref):
        s = (a_ref[0] + a_ref[1] + z_ref[...]) * dis_ref[...]
        y = s[:N, 0:2] + b_ref[...]
        m = jnp.max(y, axis=1, keepdims=True)
        e = jnp.exp(y - m)
        lse = m + jnp.log(e[:, 0:1] + e[:, 1:2])
        o_ref[...] = y - lse

    return pl.pallas_call(
        body,
        out_shape=jax.ShapeDtypeStruct((N, 2), jnp.float32),
    )(acc2, z2, dis, b2r)


_agg_wide = _make_sc_agg(D)
_agg_narrow = _make_sc_agg(CP)


def kernel(x, edge_index, W1, b1, W2, b2):
    pad = N + (jnp.arange(E_PAD - E, dtype=jnp.int32) % PAD_ROWS)
    src_p = jnp.concatenate([edge_index[0], pad])
    dst_p = jnp.concatenate([edge_index[1], pad])
    zeros1 = jnp.zeros((NP,), jnp.float32)
    zerosD = jnp.zeros((NP, D), jnp.float32)
    zerosC = jnp.zeros((NP, CP), jnp.float32)
    W2p = jnp.pad(W2, ((0, 0), (0, CP - W2.shape[1])))
    b1r = b1.reshape(1, H)
    b2r = b2.reshape(1, 2)

    deg2 = _sc_deg(dst_p, zeros1)
    g, dis = _tc_prep(x, W1, deg2.T)
    acc1 = _agg_wide(g, src_p, dst_p, zerosD)
    z2 = _tc_mid(acc1, g, dis, b1r, W2p)
    acc2 = _agg_narrow(z2, src_p, dst_p, zerosC)
    return _tc_out(acc2, z2, dis, b2r)
